# Initial kernel scaffold; baseline (speedup 1.0000x reference)
import functools

import jax
import jax.numpy as jnp
from jax import lax
from jax.experimental import pallas as pl
from jax.experimental.pallas import tpu as pltpu
from jax.experimental.pallas import tpu_sc as plsc

_ACC_W = 144
_K = 80
_NW = 32
_BLK = 400


def _tc_first(x, WT, a_l, a_r):
    n, d = x.shape
    h = WT.shape[1]

    def body(x_ref, w_ref, al_ref, ar_ref, wh_ref, s1_ref, s2_ref):
        wh = jnp.dot(x_ref[...], w_ref[...], preferred_element_type=jnp.float32)
        wh_ref[...] = wh
        s1_ref[...] = jnp.dot(wh, al_ref[...], preferred_element_type=jnp.float32)
        s2_ref[...] = jnp.dot(wh, ar_ref[...], preferred_element_type=jnp.float32)

    return pl.pallas_call(
        body,
        grid=(n // _BLK,),
        in_specs=[
            pl.BlockSpec((_BLK, d), lambda i: (i, 0)),
            pl.BlockSpec((d, h), lambda i: (0, 0)),
            pl.BlockSpec((h, 1), lambda i: (0, 0)),
            pl.BlockSpec((h, 1), lambda i: (0, 0)),
        ],
        out_specs=[
            pl.BlockSpec((_BLK, h), lambda i: (i, 0)),
            pl.BlockSpec((_BLK, 1), lambda i: (i, 0)),
            pl.BlockSpec((_BLK, 1), lambda i: (i, 0)),
        ],
        out_shape=[
            jax.ShapeDtypeStruct((n, h), jnp.float32),
            jax.ShapeDtypeStruct((n, 1), jnp.float32),
            jax.ShapeDtypeStruct((n, 1), jnp.float32),
        ],
    )(x, WT, a_l, a_r)


def _tc_mid(acc, WT, a_l, a_r):
    _, n, w = acc.shape
    h = WT.shape[1]

    def body(acc_ref, w_ref, al_ref, ar_ref, wh_ref, s1_ref, s2_ref):
        a = acc_ref[0] + acc_ref[1]
        hprev = jnp.maximum(a[:, :128] / (a[:, 128:129] + 1e-10), 0.0)
        wh = jnp.dot(hprev, w_ref[...], preferred_element_type=jnp.float32)
        wh_ref[...] = wh
        s1_ref[...] = jnp.dot(wh, al_ref[...], preferred_element_type=jnp.float32)
        s2_ref[...] = jnp.dot(wh, ar_ref[...], preferred_element_type=jnp.float32)

    return pl.pallas_call(
        body,
        grid=(n // _BLK,),
        in_specs=[
            pl.BlockSpec((2, _BLK, w), lambda i: (0, i, 0)),
            pl.BlockSpec((128, h), lambda i: (0, 0)),
            pl.BlockSpec((h, 1), lambda i: (0, 0)),
            pl.BlockSpec((h, 1), lambda i: (0, 0)),
        ],
        out_specs=[
            pl.BlockSpec((_BLK, h), lambda i: (i, 0)),
            pl.BlockSpec((_BLK, 1), lambda i: (i, 0)),
            pl.BlockSpec((_BLK, 1), lambda i: (i, 0)),
        ],
        out_shape=[
            jax.ShapeDtypeStruct((n, h), jnp.float32),
            jax.ShapeDtypeStruct((n, 1), jnp.float32),
            jax.ShapeDtypeStruct((n, 1), jnp.float32),
        ],
    )(acc, WT, a_l, a_r)


def _tc_last(acc):
    _, n, w = acc.shape

    def body(acc_ref, o_ref):
        a = acc_ref[0] + acc_ref[1]
        o_ref[...] = jnp.maximum(a[:, :128] / (a[:, 128:129] + 1e-10), 0.0)

    return pl.pallas_call(
        body,
        grid=(n // _BLK,),
        in_specs=[pl.BlockSpec((2, _BLK, w), lambda i: (0, i, 0))],
        out_specs=pl.BlockSpec((_BLK, 128), lambda i: (i, 0)),
        out_shape=jax.ShapeDtypeStruct((n, 128), jnp.float32),
    )(acc)


def _sc_edge_pass(Wh, s1, s2, srcr, dstr):
    n = Wh.shape[0]
    nchunks = srcr.shape[1]
    rows_per_sub = n // 16
    ncopy = rows_per_sub // 125
    mesh = plsc.VectorSubcoreMesh(core_axis_name="c", subcore_axis_name="s")

    @functools.partial(
        pl.kernel,
        out_type=jax.ShapeDtypeStruct((2, n, _ACC_W), jnp.float32),
        mesh=mesh,
        scratch_types=[
            pltpu.VMEM((n,), jnp.float32),
            pltpu.VMEM((n,), jnp.float32),
            pltpu.VMEM((nchunks, _K), jnp.int32),
            pltpu.VMEM((nchunks, _K), jnp.int32),
            pltpu.VMEM((_K,), jnp.float32),
            pltpu.VMEM((_K, 128), jnp.float32),
            pltpu.VMEM((_K, _ACC_W), jnp.float32),
            pltpu.VMEM((125, _ACC_W), jnp.float32),
            pltpu.VMEM_SHARED((n, _ACC_W), jnp.float32),
            pltpu.SemaphoreType.DMA,
        ],
    )
    def k(wh_hbm, s1_hbm, s2_hbm, src_hbm, dst_hbm, out_hbm,
          s1_v, s2_v, src_v, dst_v, p_v, rg, ro, zb, acc, sem):
        cid = lax.axis_index("c")
        sid = lax.axis_index("s")
        wid = cid * 16 + sid
        pltpu.sync_copy(s1_hbm, s1_v)
        pltpu.sync_copy(s2_hbm, s2_v)
        pltpu.sync_copy(src_hbm.at[wid], src_v)
        pltpu.sync_copy(dst_hbm.at[wid], dst_v)

        def zb_body(i, c):
            for t in range(_ACC_W // 16):
                zb[i, pl.ds(t * 16, 16)] = jnp.zeros((16,), jnp.float32)
            return c

        lax.fori_loop(0, 125, zb_body, 0)
        base = sid * rows_per_sub
        for q in range(ncopy):
            pltpu.sync_copy(zb, acc.at[pl.ds(base + q * 125, 125)])
        plsc.subcore_barrier()

        lane0 = lax.iota(jnp.int32, 16) == 0

        def chunk(j, c):
            pltpu.async_copy(wh_hbm.at[dst_v.at[j]], rg, sem).wait()
            for t in range(_K // 16):
                sv = src_v[j, pl.ds(t * 16, 16)]
                dv = dst_v[j, pl.ds(t * 16, 16)]
                z = plsc.load_gather(s1_v, [sv]) + plsc.load_gather(s2_v, [dv])
                p_v[pl.ds(t * 16, 16)] = jnp.exp(jnp.minimum(-z, -0.2 * z))

            def row(i, c2):
                ps = p_v[i]
                for t in range(8):
                    ro[i, pl.ds(t * 16, 16)] = rg[i, pl.ds(t * 16, 16)] * ps
                ro[i, pl.ds(128, 16)] = jnp.where(lane0, ps, 0.0)
                return c2

            lax.fori_loop(0, _K, row, 0)
            pltpu.sync_copy(ro, acc.at[src_v.at[j]], add=True)
            return c

        lax.fori_loop(0, nchunks, chunk, 0)
        plsc.subcore_barrier()
        for q in range(ncopy):
            sl = pl.ds(base + q * 125, 125)
            pltpu.sync_copy(acc.at[sl], out_hbm.at[cid].at[sl])

    return k(Wh, s1, s2, srcr, dstr)


@jax.jit
def kernel(x, edge_index, W1, a1, W2, a2):
    n, d = x.shape
    e = edge_index.shape[1]
    h = W1.shape[0]
    o = W2.shape[0]
    src = edge_index[0].reshape(_NW, e // _NW // _K, _K)
    dst = edge_index[1].reshape(_NW, e // _NW // _K, _K)

    Wh1, s1, s2 = _tc_first(x, W1.T, a1[:, :h].T, a1[:, h:].T)
    acc1 = _sc_edge_pass(Wh1, s1.reshape(n), s2.reshape(n), src, dst)
    Wh2, s1b, s2b = _tc_mid(acc1, W2.T, a2[:, :o].T, a2[:, o:].T)
    acc2 = _sc_edge_pass(Wh2, s1b.reshape(n), s2b.reshape(n), src, dst)
    return _tc_last(acc2)


# speedup vs baseline: 2.3382x; 2.3382x over previous
import functools

import jax
import jax.numpy as jnp
from jax import lax
from jax.experimental import pallas as pl
from jax.experimental.pallas import tpu as pltpu
from jax.experimental.pallas import tpu_sc as plsc

_K = 128
_NW = 16
_BLK = 400
_STRIP = 80
_WIN = 2560


def _tc_first(x, WT, a_l, a_r):
    n, d = x.shape
    h = WT.shape[1]

    def body(x_ref, w_ref, al_ref, ar_ref, wh_ref, s1_ref, s2_ref):
        wh = jnp.dot(x_ref[...], w_ref[...], preferred_element_type=jnp.float32)
        wh_ref[...] = wh
        s1_ref[...] = jnp.dot(wh, al_ref[...], preferred_element_type=jnp.float32)
        s2_ref[...] = jnp.dot(wh, ar_ref[...], preferred_element_type=jnp.float32)

    return pl.pallas_call(
        body,
        grid=(n // _BLK,),
        in_specs=[
            pl.BlockSpec((_BLK, d), lambda i: (i, 0)),
            pl.BlockSpec((d, h), lambda i: (0, 0)),
            pl.BlockSpec((h, 1), lambda i: (0, 0)),
            pl.BlockSpec((h, 1), lambda i: (0, 0)),
        ],
        out_specs=[
            pl.BlockSpec((_BLK, h), lambda i: (i, 0)),
            pl.BlockSpec((_BLK, 1), lambda i: (i, 0)),
            pl.BlockSpec((_BLK, 1), lambda i: (i, 0)),
        ],
        out_shape=[
            jax.ShapeDtypeStruct((n, h), jnp.float32),
            jax.ShapeDtypeStruct((n, 1), jnp.float32),
            jax.ShapeDtypeStruct((n, 1), jnp.float32),
        ],
    )(x, WT, a_l, a_r)


def _tc_post(acc, s_all):
    n, d = acc.shape
    nw = s_all.shape[0]
    s_t = s_all.T.reshape(n // _BLK, _BLK, nw)

    def body(acc_ref, s_ref, o_ref):
        num = acc_ref[...]
        den = jnp.sum(s_ref[0], axis=1)[:, None] + 1e-10
        o_ref[...] = jnp.maximum(num / den, 0.0)

    return pl.pallas_call(
        body,
        grid=(n // _BLK,),
        in_specs=[
            pl.BlockSpec((_BLK, d), lambda i: (i, 0)),
            pl.BlockSpec((1, _BLK, nw), lambda i: (i, 0, 0)),
        ],
        out_specs=pl.BlockSpec((_BLK, d), lambda i: (i, 0)),
        out_shape=jax.ShapeDtypeStruct((n, d), jnp.float32),
    )(acc, s_t)


def _sc_edge_pass(Wh, s1, s2, srcr, dstr, ntailg):
    n, d = Wh.shape
    nchunks = srcr.shape[1]
    nfull = nchunks - 1
    nwin = (n + _WIN - 1) // _WIN
    mesh = plsc.VectorSubcoreMesh(
        core_axis_name="c", subcore_axis_name="s", num_cores=1)

    @functools.partial(
        pl.kernel,
        out_type=(
            jax.ShapeDtypeStruct((n, d), jnp.float32),
            jax.ShapeDtypeStruct((_NW, n), jnp.float32),
        ),
        mesh=mesh,
        compiler_params=pltpu.CompilerParams(needs_layout_passes=False),
        scratch_types=[
            pltpu.VMEM((n,), jnp.float32),
            pltpu.VMEM((n,), jnp.float32),
            pltpu.VMEM((n,), jnp.float32),
            pltpu.VMEM((nchunks, _K), jnp.int32),
            pltpu.VMEM((nchunks, _K), jnp.int32),
            pltpu.VMEM((1, _K), jnp.int32),
            pltpu.VMEM((1, 16 * ntailg), jnp.int32),
            pltpu.VMEM((_K, 128), jnp.float32),
            pltpu.VMEM((16,), jnp.int32),
            pltpu.VMEM((16,), jnp.float32),
            pltpu.VMEM_SHARED((_WIN, 128), jnp.float32),
            pltpu.SemaphoreType.DMA,
        ],
    )
    def k(wh_hbm, s1_hbm, s2_hbm, src_hbm, dst_hbm, outh_hbm, outs_hbm,
          s1_v, s2_v, sv_v, src_v, dst_v, scl_v, scl2_v, rg, tks, tcs,
          acc, sem):
        sid = lax.axis_index("s")
        wid = sid
        pltpu.sync_copy(s1_hbm, s1_v)
        pltpu.sync_copy(s2_hbm, s2_v)
        pltpu.sync_copy(src_hbm.at[wid], src_v)
        pltpu.sync_copy(dst_hbm.at[wid], dst_v)

        zeros16 = jnp.zeros((16,), jnp.float32)

        def sv_zero(i, c):
            sv_v[pl.ds(pl.multiple_of(i * 16, 8), 16)] = zeros16
            return c

        lax.fori_loop(0, n // 16, sv_zero, 0)

        def rg_zero(i, c):
            for t in range(8):
                rg[i, pl.ds(t * 16, 16)] = zeros16
            return c

        iota = lax.iota(jnp.int32, 16)
        ip = jnp.minimum(iota + 1, 15)
        im = jnp.maximum(iota - 1, 0)

        def make_group(w, lo, compute_s, idx_ref):
            def group(j, t16):
                sv = src_v[j, pl.ds(t16, 16)]
                dv = dst_v[j, pl.ds(t16, 16)]
                z = (plsc.load_gather(s1_v, [sv])
                     + plsc.load_gather(s2_v, [dv]))
                p = jnp.exp(jnp.minimum(-z, -0.2 * z))
                if compute_s:
                    ks, vs = plsc.sort_key_val(sv, p)
                    cs = plsc.cumsum(vs)
                    tks[...] = ks
                    tcs[...] = cs
                    ks_next = plsc.load_gather(tks, [ip])
                    ks_prev = plsc.load_gather(tks, [im])
                    cs_prev = plsc.load_gather(tcs, [im])
                    is_end = (ks != ks_next) | (iota == 15)
                    is_start = (ks != ks_prev) & (iota > 0)
                    plsc.addupdate_scatter(sv_v, [ks], cs, mask=is_end)
                    plsc.addupdate_scatter(sv_v, [ks], -cs_prev,
                                           mask=is_start)
                loc = sv - lo
                valid = (loc >= 0) & (loc < _WIN)
                p = jnp.where(valid, p, 0.0)
                idx_ref[0, pl.ds(t16, 16)] = jnp.clip(loc, 0, _WIN - 1)
                for r in range(16):
                    ps = p[r]
                    i = t16 + r
                    for u in range(8):
                        rg[i, pl.ds(u * 16, 16)] = rg[i, pl.ds(u * 16, 16)] * ps
            return group

        for w in range(nwin):
            lo = w * _WIN
            wrows = min(n - lo, _WIN)
            lax.fori_loop(0, _K, rg_zero, 0)
            nz = _WIN // _K
            for q in range((nz + 15) // 16):
                idx = sid + 16 * q

                @pl.when(idx < nz)
                def _():
                    start = pl.multiple_of(idx * _K, 8)
                    pltpu.sync_copy(rg, acc.at[pl.ds(start, _K)])

            plsc.subcore_barrier()

            group = make_group(w, lo, w == 0, scl_v)
            groupt = make_group(w, lo, w == 0, scl2_v)

            def chunk(j, c):
                pltpu.async_copy(wh_hbm.at[dst_v.at[j]], rg, sem).wait()

                def tbody(t, c2):
                    group(j, pl.multiple_of(t * 16, 16))
                    return c2

                lax.fori_loop(0, _K // 16, tbody, 0)
                pltpu.sync_copy(rg, acc.at[scl_v.at[0]], add=True)
                return c

            lax.fori_loop(0, nfull, chunk, 0)

            pltpu.async_copy(wh_hbm.at[dst_v.at[nfull]], rg, sem).wait()
            for t in range(ntailg):
                groupt(nfull, t * 16)
            pltpu.sync_copy(rg.at[pl.ds(0, 16 * ntailg)],
                            acc.at[scl2_v.at[0]], add=True)

            plsc.subcore_barrier()
            ndump = wrows // _STRIP
            for q in range((ndump + 15) // 16):
                idx = sid + 16 * q

                @pl.when(idx < ndump)
                def _():
                    start = pl.multiple_of(idx * _STRIP, 8)
                    pltpu.sync_copy(
                        acc.at[pl.ds(start, _STRIP)],
                        outh_hbm.at[pl.ds(pl.multiple_of(lo, 8) + start,
                                          _STRIP)])

            plsc.subcore_barrier()

        pltpu.sync_copy(sv_v, outs_hbm.at[wid])

    return k(Wh, s1, s2, srcr, dstr)


@jax.jit
def kernel(x, edge_index, W1, a1, W2, a2):
    n, d = x.shape
    e = edge_index.shape[1]
    h = W1.shape[0]
    o = W2.shape[0]
    eper = e // _NW
    nfull = eper // _K
    ntail = eper - nfull * _K
    ntailg = ntail // 16
    pad = _K - ntail
    src = jnp.pad(edge_index[0].reshape(_NW, eper), ((0, 0), (0, pad)))
    dst = jnp.pad(edge_index[1].reshape(_NW, eper), ((0, 0), (0, pad)))
    src = src.reshape(_NW, nfull + 1, _K)
    dst = dst.reshape(_NW, nfull + 1, _K)

    wt_stack = jnp.stack([W1.T, W2.T])
    al_stack = jnp.stack([a1[:, :h].T, a2[:, :o].T])
    ar_stack = jnp.stack([a1[:, h:].T, a2[:, o:].T])

    def layer_step(hcur, ws):
        wt, al, ar = ws
        Wh, s1, s2 = _tc_first(hcur, wt, al, ar)
        acc, sall = _sc_edge_pass(Wh, s1.reshape(n), s2.reshape(n), src, dst,
                                  ntailg)
        return _tc_post(acc, sall), None

    hfinal, _ = lax.scan(layer_step, x, (wt_stack, al_stack, ar_stack))
    return hfinal


# speedup vs baseline: 2.5047x; 1.0712x over previous
import functools

import jax
import jax.numpy as jnp
from jax import lax
from jax.experimental import pallas as pl
from jax.experimental.pallas import tpu as pltpu
from jax.experimental.pallas import tpu_sc as plsc

_K = 128
_NW = 16
_BLK = 400
_STRIP = 80
_WIN = 2000


def _tc_first(x, WT, a_l, a_r):
    n, d = x.shape
    h = WT.shape[1]

    def body(x_ref, w_ref, al_ref, ar_ref, wh_ref, s1_ref, s2_ref):
        wh = jnp.dot(x_ref[...], w_ref[...], preferred_element_type=jnp.float32)
        wh_ref[...] = wh
        s1_ref[...] = jnp.dot(wh, al_ref[...], preferred_element_type=jnp.float32)
        s2_ref[...] = jnp.dot(wh, ar_ref[...], preferred_element_type=jnp.float32)

    return pl.pallas_call(
        body,
        grid=(n // _BLK,),
        in_specs=[
            pl.BlockSpec((_BLK, d), lambda i: (i, 0)),
            pl.BlockSpec((d, h), lambda i: (0, 0)),
            pl.BlockSpec((h, 1), lambda i: (0, 0)),
            pl.BlockSpec((h, 1), lambda i: (0, 0)),
        ],
        out_specs=[
            pl.BlockSpec((_BLK, h), lambda i: (i, 0)),
            pl.BlockSpec((_BLK, 1), lambda i: (i, 0)),
            pl.BlockSpec((_BLK, 1), lambda i: (i, 0)),
        ],
        out_shape=[
            jax.ShapeDtypeStruct((n, h), jnp.float32),
            jax.ShapeDtypeStruct((n, 1), jnp.float32),
            jax.ShapeDtypeStruct((n, 1), jnp.float32),
        ],
    )(x, WT, a_l, a_r)


def _tc_post(acc, s_all):
    n, d = acc.shape
    nw = s_all.shape[0]
    s_t = s_all.T.reshape(n // _BLK, _BLK, nw)

    def body(acc_ref, s_ref, o_ref):
        num = acc_ref[...]
        den = jnp.sum(s_ref[0], axis=1)[:, None] + 1e-10
        o_ref[...] = jnp.maximum(num / den, 0.0)

    return pl.pallas_call(
        body,
        grid=(n // _BLK,),
        in_specs=[
            pl.BlockSpec((_BLK, d), lambda i: (i, 0)),
            pl.BlockSpec((1, _BLK, nw), lambda i: (i, 0, 0)),
        ],
        out_specs=pl.BlockSpec((_BLK, d), lambda i: (i, 0)),
        out_shape=jax.ShapeDtypeStruct((n, d), jnp.float32),
    )(acc, s_t)


def _sc_edge_pass(Wh, s1, s2, srcr, dstr, ntailg):
    n, d = Wh.shape
    nchunks = srcr.shape[1]
    nfull = nchunks - 1
    nwin = (n + _WIN - 1) // _WIN
    mesh = plsc.VectorSubcoreMesh(
        core_axis_name="c", subcore_axis_name="s", num_cores=1)

    @functools.partial(
        pl.kernel,
        out_type=(
            jax.ShapeDtypeStruct((n, d), jnp.float32),
            jax.ShapeDtypeStruct((_NW, n), jnp.float32),
        ),
        mesh=mesh,
        compiler_params=pltpu.CompilerParams(needs_layout_passes=False),
        scratch_types=[
            pltpu.VMEM((n,), jnp.float32),
            pltpu.VMEM((n,), jnp.float32),
            pltpu.VMEM((n,), jnp.float32),
            pltpu.VMEM((nchunks, _K), jnp.int32),
            pltpu.VMEM((nchunks, _K), jnp.int32),
            pltpu.VMEM((1, _K), jnp.int32),
            pltpu.VMEM((1, _K), jnp.int32),
            pltpu.VMEM((1, 16 * ntailg), jnp.int32),
            pltpu.VMEM((_K, 128), jnp.float32),
            pltpu.VMEM((_K, 128), jnp.float32),
            pltpu.VMEM((16,), jnp.int32),
            pltpu.VMEM((16,), jnp.float32),
            pltpu.VMEM_SHARED((_WIN, 128), jnp.float32),
            pltpu.SemaphoreType.DMA,
            pltpu.SemaphoreType.DMA,
        ],
    )
    def k(wh_hbm, s1_hbm, s2_hbm, src_hbm, dst_hbm, outh_hbm, outs_hbm,
          s1_v, s2_v, sv_v, src_v, dst_v, scl0_v, scl1_v, scl2_v, rg0, rg1,
          tks, tcs, acc, sem0, sem1):
        sid = lax.axis_index("s")
        wid = sid
        pltpu.sync_copy(s1_hbm, s1_v)
        pltpu.sync_copy(s2_hbm, s2_v)
        pltpu.sync_copy(src_hbm.at[wid], src_v)
        pltpu.sync_copy(dst_hbm.at[wid], dst_v)

        zeros16 = jnp.zeros((16,), jnp.float32)

        def sv_zero(i, c):
            sv_v[pl.ds(pl.multiple_of(i * 16, 8), 16)] = zeros16
            return c

        lax.fori_loop(0, n // 16, sv_zero, 0)

        def rg_zero(i, c):
            for t in range(8):
                rg0[i, pl.ds(t * 16, 16)] = zeros16
            return c

        iota = lax.iota(jnp.int32, 16)
        ip = jnp.minimum(iota + 1, 15)
        im = jnp.maximum(iota - 1, 0)

        def make_group(w, lo, compute_s, idx_ref, rg):
            def group(j, t16):
                sv = src_v[j, pl.ds(t16, 16)]
                dv = dst_v[j, pl.ds(t16, 16)]
                z = (plsc.load_gather(s1_v, [sv])
                     + plsc.load_gather(s2_v, [dv]))
                p = jnp.exp(jnp.minimum(-z, -0.2 * z))
                if compute_s:
                    ks, vs = plsc.sort_key_val(sv, p)
                    cs = plsc.cumsum(vs)
                    tks[...] = ks
                    tcs[...] = cs
                    ks_next = plsc.load_gather(tks, [ip])
                    ks_prev = plsc.load_gather(tks, [im])
                    cs_prev = plsc.load_gather(tcs, [im])
                    is_end = (ks != ks_next) | (iota == 15)
                    is_start = (ks != ks_prev) & (iota > 0)
                    plsc.addupdate_scatter(sv_v, [ks], cs, mask=is_end)
                    plsc.addupdate_scatter(sv_v, [ks], -cs_prev,
                                           mask=is_start)
                loc = sv - lo
                valid = (loc >= 0) & (loc < _WIN)
                p = jnp.where(valid, p, 0.0)
                idx_ref[0, pl.ds(t16, 16)] = jnp.clip(loc, 0, _WIN - 1)
                for r in range(16):
                    ps = p[r]
                    i = t16 + r
                    for u in range(8):
                        rg[i, pl.ds(u * 16, 16)] = rg[i, pl.ds(u * 16, 16)] * ps
            return group

        npairs = nfull // 2

        for w in range(nwin):
            lo = w * _WIN
            wrows = min(n - lo, _WIN)
            lax.fori_loop(0, _STRIP, rg_zero, 0)
            nz = _WIN // _STRIP
            for q in range((nz + 15) // 16):
                idx = sid + 16 * q

                @pl.when(idx < nz)
                def _():
                    start = pl.multiple_of(idx * _STRIP, 8)
                    pltpu.sync_copy(rg0.at[pl.ds(0, _STRIP)],
                                    acc.at[pl.ds(start, _STRIP)])

            plsc.subcore_barrier()

            group0 = make_group(w, lo, w == 0, scl0_v, rg0)
            group1 = make_group(w, lo, w == 0, scl1_v, rg1)
            groupt = make_group(w, lo, w == 0, scl2_v, rg0)

            def compute(j, group):
                def tbody(t, c2):
                    group(j, pl.multiple_of(t * 16, 16))
                    return c2

                lax.fori_loop(0, _K // 16, tbody, 0)

            pltpu.async_copy(wh_hbm.at[dst_v.at[0]], rg0, sem0)
            pltpu.async_copy(wh_hbm.at[dst_v.at[1]], rg1, sem1)

            def pair(jj, c):
                j0 = jj * 2
                pltpu.make_async_copy(wh_hbm.at[dst_v.at[j0]], rg0,
                                      sem0).wait()
                compute(j0, group0)
                pltpu.sync_copy(rg0, acc.at[scl0_v.at[0]], add=True)

                @pl.when(jj + 1 < npairs)
                def _():
                    pltpu.async_copy(wh_hbm.at[dst_v.at[j0 + 2]], rg0, sem0)

                pltpu.make_async_copy(wh_hbm.at[dst_v.at[j0 + 1]], rg1,
                                      sem1).wait()
                compute(j0 + 1, group1)
                pltpu.sync_copy(rg1, acc.at[scl1_v.at[0]], add=True)

                @pl.when(jj + 1 < npairs)
                def _():
                    pltpu.async_copy(wh_hbm.at[dst_v.at[j0 + 3]], rg1, sem1)

                return c

            lax.fori_loop(0, npairs, pair, 0)

            pltpu.async_copy(wh_hbm.at[dst_v.at[nfull]], rg0, sem0).wait()
            for t in range(ntailg):
                groupt(nfull, t * 16)
            pltpu.sync_copy(rg0.at[pl.ds(0, 16 * ntailg)],
                            acc.at[scl2_v.at[0]], add=True)

            plsc.subcore_barrier()
            ndump = wrows // _STRIP
            for q in range((ndump + 15) // 16):
                idx = sid + 16 * q

                @pl.when(idx < ndump)
                def _():
                    start = pl.multiple_of(idx * _STRIP, 8)
                    pltpu.sync_copy(
                        acc.at[pl.ds(start, _STRIP)],
                        outh_hbm.at[pl.ds(pl.multiple_of(lo, 8) + start,
                                          _STRIP)])

            plsc.subcore_barrier()

        pltpu.sync_copy(sv_v, outs_hbm.at[wid])

    return k(Wh, s1, s2, srcr, dstr)


@jax.jit
def kernel(x, edge_index, W1, a1, W2, a2):
    n, d = x.shape
    e = edge_index.shape[1]
    h = W1.shape[0]
    o = W2.shape[0]
    eper = e // _NW
    nfull = eper // _K
    ntail = eper - nfull * _K
    ntailg = ntail // 16
    pad = _K - ntail
    src = jnp.pad(edge_index[0].reshape(_NW, eper), ((0, 0), (0, pad)))
    dst = jnp.pad(edge_index[1].reshape(_NW, eper), ((0, 0), (0, pad)))
    src = src.reshape(_NW, nfull + 1, _K)
    dst = dst.reshape(_NW, nfull + 1, _K)

    wt_stack = jnp.stack([W1.T, W2.T])
    al_stack = jnp.stack([a1[:, :h].T, a2[:, :o].T])
    ar_stack = jnp.stack([a1[:, h:].T, a2[:, o:].T])

    def layer_step(hcur, ws):
        wt, al, ar = ws
        Wh, s1, s2 = _tc_first(hcur, wt, al, ar)
        acc, sall = _sc_edge_pass(Wh, s1.reshape(n), s2.reshape(n), src, dst,
                                  ntailg)
        return _tc_post(acc, sall), None

    hfinal, _ = lax.scan(layer_step, x, (wt_stack, al_stack, ar_stack))
    return hfinal


# speedup vs baseline: 3.8153x; 1.5232x over previous
import functools

import jax
import jax.numpy as jnp
from jax import lax
from jax.experimental import pallas as pl
from jax.experimental.pallas import tpu as pltpu
from jax.experimental.pallas import tpu_sc as plsc

_K = 128
_NW = 16
_BLK = 400
_STRIP = 80
_WIN = 4000


def _tc_first(x, WT, a_l, a_r):
    n, d = x.shape
    h = WT.shape[1]

    def body(x_ref, w_ref, al_ref, ar_ref, wh_ref, s1_ref, s2_ref):
        wh = jnp.dot(x_ref[...], w_ref[...], preferred_element_type=jnp.float32)
        wh_ref[...] = wh
        s1_ref[...] = jnp.dot(wh, al_ref[...], preferred_element_type=jnp.float32)
        s2_ref[...] = jnp.dot(wh, ar_ref[...], preferred_element_type=jnp.float32)

    return pl.pallas_call(
        body,
        grid=(n // _BLK,),
        in_specs=[
            pl.BlockSpec((_BLK, d), lambda i: (i, 0)),
            pl.BlockSpec((d, h), lambda i: (0, 0)),
            pl.BlockSpec((h, 1), lambda i: (0, 0)),
            pl.BlockSpec((h, 1), lambda i: (0, 0)),
        ],
        out_specs=[
            pl.BlockSpec((_BLK, h), lambda i: (i, 0)),
            pl.BlockSpec((_BLK, 1), lambda i: (i, 0)),
            pl.BlockSpec((_BLK, 1), lambda i: (i, 0)),
        ],
        out_shape=[
            jax.ShapeDtypeStruct((n, h), jnp.float32),
            jax.ShapeDtypeStruct((n, 1), jnp.float32),
            jax.ShapeDtypeStruct((n, 1), jnp.float32),
        ],
    )(x, WT, a_l, a_r)


def _tc_post(acc, s_all):
    n, d = acc.shape
    nw = s_all.shape[0]
    s_t = s_all.T.reshape(n // _BLK, _BLK, nw)

    def body(acc_ref, s_ref, o_ref):
        num = acc_ref[...]
        den = jnp.sum(s_ref[0], axis=1)[:, None] + 1e-10
        o_ref[...] = jnp.maximum(num / den, 0.0)

    return pl.pallas_call(
        body,
        grid=(n // _BLK,),
        in_specs=[
            pl.BlockSpec((_BLK, d), lambda i: (i, 0)),
            pl.BlockSpec((1, _BLK, nw), lambda i: (i, 0, 0)),
        ],
        out_specs=pl.BlockSpec((_BLK, d), lambda i: (i, 0)),
        out_shape=jax.ShapeDtypeStruct((n, d), jnp.float32),
    )(acc, s_t)


def _sc_edge_pass(Wh, s1, s2, srcr, dstr, ntailg):
    n, d = Wh.shape
    nchunks = srcr.shape[1] * 2 // _K
    nfull = nchunks - 1
    nwin = (n + _WIN - 1) // _WIN
    mesh = plsc.VectorSubcoreMesh(
        core_axis_name="c", subcore_axis_name="s", num_cores=1)

    @functools.partial(
        pl.kernel,
        out_type=(
            jax.ShapeDtypeStruct((n, d), jnp.float32),
            jax.ShapeDtypeStruct((_NW, n), jnp.float32),
        ),
        mesh=mesh,
        compiler_params=pltpu.CompilerParams(needs_layout_passes=False),
        scratch_types=[
            pltpu.VMEM((n,), jnp.float32),
            pltpu.VMEM((n,), jnp.float32),
            pltpu.VMEM((n,), jnp.float32),
            pltpu.VMEM((nchunks * _K // 2,), jnp.int32),
            pltpu.VMEM((nchunks * _K // 2,), jnp.int32),
            pltpu.VMEM((1, _K), jnp.int32),
            pltpu.VMEM((1, _K), jnp.int32),
            pltpu.VMEM((1, _K), jnp.int32),
            pltpu.VMEM((1, _K), jnp.int32),
            pltpu.VMEM((1, 16 * ntailg), jnp.int32),
            pltpu.VMEM((_K, 128), jnp.float32),
            pltpu.VMEM((_K, 128), jnp.float32),
            pltpu.VMEM((16,), jnp.int32),
            pltpu.VMEM((16,), jnp.float32),
            pltpu.VMEM_SHARED((_WIN, 128), jnp.float32),
            pltpu.SemaphoreType.DMA,
            pltpu.SemaphoreType.DMA,
        ],
    )
    def k(wh_hbm, s1_hbm, s2_hbm, src_hbm, dst_hbm, outh_hbm, outs_hbm,
          s1_v, s2_v, sv_v, src_v, dst_v, dstw0, dstw1, scl0_v, scl1_v,
          scl2_v, rg0, rg1, tks, tcs, acc, sem0, sem1):
        sid = lax.axis_index("s")
        wid = sid
        pltpu.sync_copy(s1_hbm, s1_v)
        pltpu.sync_copy(s2_hbm, s2_v)
        pltpu.sync_copy(src_hbm.at[wid], src_v)
        pltpu.sync_copy(dst_hbm.at[wid], dst_v)

        zeros16 = jnp.zeros((16,), jnp.float32)

        def sv_zero(i, c):
            sv_v[pl.ds(pl.multiple_of(i * 16, 8), 16)] = zeros16
            return c

        lax.fori_loop(0, n // 16, sv_zero, 0)

        def rg_zero(i, c):
            for t in range(8):
                rg0[i, pl.ds(t * 16, 16)] = zeros16
            return c

        iota = lax.iota(jnp.int32, 16)
        ip = jnp.minimum(iota + 1, 15)
        im = jnp.maximum(iota - 1, 0)

        def build_dstw(j, dstw):
            for t in range(_K // 32):
                dw = dst_v[pl.ds(pl.multiple_of(j * (_K // 2) + t * 16, 16),
                                 16)]
                d32 = plsc.bitcast(dw, jnp.int16)
                da, db = plsc.unpack(d32, format=plsc.PackFormat.INTERLEAVED,
                                     preferred_element_type=jnp.int32)
                dstw[0, pl.ds(t * 32, 16)] = da
                dstw[0, pl.ds(t * 32 + 16, 16)] = db

        def make_group(w, lo, compute_s, idx_ref, rg, dstw):
            def group(j, g32):
                sw = src_v[pl.ds(pl.multiple_of(j * (_K // 2) + g32 // 2, 16),
                                 16)]
                s32 = plsc.bitcast(sw, jnp.int16)
                sva, svb = plsc.unpack(
                    s32, format=plsc.PackFormat.INTERLEAVED,
                    preferred_element_type=jnp.int32)
                for half, sv in ((0, sva), (1, svb)):
                    t16 = g32 + 16 * half
                    dv = dstw[0, pl.ds(t16, 16)]
                    z = (plsc.load_gather(s1_v, [sv])
                         + plsc.load_gather(s2_v, [dv]))
                    p = jnp.exp(jnp.minimum(-z, -0.2 * z))
                    if compute_s:
                        ks, vs = plsc.sort_key_val(sv, p)
                        cs = plsc.cumsum(vs)
                        tks[...] = ks
                        tcs[...] = cs
                        ks_next = plsc.load_gather(tks, [ip])
                        ks_prev = plsc.load_gather(tks, [im])
                        cs_prev = plsc.load_gather(tcs, [im])
                        is_end = (ks != ks_next) | (iota == 15)
                        is_start = (ks != ks_prev) & (iota > 0)
                        plsc.addupdate_scatter(sv_v, [ks], cs, mask=is_end)
                        plsc.addupdate_scatter(sv_v, [ks], -cs_prev,
                                               mask=is_start)
                    loc = sv - lo
                    valid = (loc >= 0) & (loc < _WIN)
                    p = jnp.where(valid, p, 0.0)
                    idx_ref[0, pl.ds(t16, 16)] = jnp.clip(loc, 0, _WIN - 1)
                    for r in range(16):
                        ps = p[r]
                        i = t16 + r
                        for u in range(8):
                            rg[i, pl.ds(u * 16, 16)] = (
                                rg[i, pl.ds(u * 16, 16)] * ps)
            return group

        npairs = nfull // 2

        for w in range(nwin):
            lo = w * _WIN
            wrows = min(n - lo, _WIN)
            lax.fori_loop(0, _STRIP, rg_zero, 0)
            nz = _WIN // _STRIP
            for q in range((nz + 15) // 16):
                idx = sid + 16 * q

                @pl.when(idx < nz)
                def _():
                    start = pl.multiple_of(idx * _STRIP, 8)
                    pltpu.sync_copy(rg0.at[pl.ds(0, _STRIP)],
                                    acc.at[pl.ds(start, _STRIP)])

            plsc.subcore_barrier()

            group0 = make_group(w, lo, w == 0, scl0_v, rg0, dstw0)
            group1 = make_group(w, lo, w == 0, scl1_v, rg1, dstw1)
            groupt = make_group(w, lo, w == 0, scl2_v, rg0, dstw0)

            def compute(j, group):
                def tbody(t, c2):
                    group(j, pl.multiple_of(t * 32, 32))
                    return c2

                lax.fori_loop(0, _K // 32, tbody, 0)

            build_dstw(0, dstw0)
            pltpu.async_copy(wh_hbm.at[dstw0.at[0]], rg0, sem0)
            build_dstw(1, dstw1)
            pltpu.async_copy(wh_hbm.at[dstw1.at[0]], rg1, sem1)

            def pair(jj, c):
                j0 = jj * 2
                pltpu.make_async_copy(wh_hbm.at[dstw0.at[0]], rg0,
                                      sem0).wait()
                compute(j0, group0)
                pltpu.sync_copy(rg0, acc.at[scl0_v.at[0]], add=True)

                @pl.when(jj + 1 < npairs)
                def _():
                    build_dstw(j0 + 2, dstw0)
                    pltpu.async_copy(wh_hbm.at[dstw0.at[0]], rg0, sem0)

                pltpu.make_async_copy(wh_hbm.at[dstw1.at[0]], rg1,
                                      sem1).wait()
                compute(j0 + 1, group1)
                pltpu.sync_copy(rg1, acc.at[scl1_v.at[0]], add=True)

                @pl.when(jj + 1 < npairs)
                def _():
                    build_dstw(j0 + 3, dstw1)
                    pltpu.async_copy(wh_hbm.at[dstw1.at[0]], rg1, sem1)

                return c

            lax.fori_loop(0, npairs, pair, 0)

            build_dstw(nfull, dstw0)
            pltpu.async_copy(wh_hbm.at[dstw0.at[0]], rg0, sem0).wait()
            for t in range(ntailg // 2):
                groupt(nfull, t * 32)
            pltpu.sync_copy(rg0.at[pl.ds(0, 16 * ntailg)],
                            acc.at[scl2_v.at[0]], add=True)

            plsc.subcore_barrier()
            ndump = wrows // _STRIP
            for q in range((ndump + 15) // 16):
                idx = sid + 16 * q

                @pl.when(idx < ndump)
                def _():
                    start = pl.multiple_of(idx * _STRIP, 8)
                    pltpu.sync_copy(
                        acc.at[pl.ds(start, _STRIP)],
                        outh_hbm.at[pl.ds(pl.multiple_of(lo, 8) + start,
                                          _STRIP)])

            plsc.subcore_barrier()

        pltpu.sync_copy(sv_v, outs_hbm.at[wid])

    return k(Wh, s1, s2, srcr, dstr)


@jax.jit
def kernel(x, edge_index, W1, a1, W2, a2):
    n, d = x.shape
    e = edge_index.shape[1]
    h = W1.shape[0]
    o = W2.shape[0]
    eper = e // _NW
    nfull = eper // _K
    ntail = eper - nfull * _K
    ntailg = ntail // 16
    pad = _K - ntail
    ei16 = edge_index.astype(jnp.int16)
    src = jnp.pad(ei16[0].reshape(_NW, eper), ((0, 0), (0, pad)))
    dst = jnp.pad(ei16[1].reshape(_NW, eper), ((0, 0), (0, pad)))
    src = lax.bitcast_convert_type(src.reshape(_NW, -1, 2), jnp.int32)
    dst = lax.bitcast_convert_type(dst.reshape(_NW, -1, 2), jnp.int32)

    wt_stack = jnp.stack([W1.T, W2.T])
    al_stack = jnp.stack([a1[:, :h].T, a2[:, :o].T])
    ar_stack = jnp.stack([a1[:, h:].T, a2[:, o:].T])

    def layer_step(hcur, ws):
        wt, al, ar = ws
        Wh, s1, s2 = _tc_first(hcur, wt, al, ar)
        acc, sall = _sc_edge_pass(Wh, s1.reshape(n), s2.reshape(n), src, dst,
                                  ntailg)
        return _tc_post(acc, sall), None

    hfinal, _ = lax.scan(layer_step, x, (wt_stack, al_stack, ar_stack))
    return hfinal


# speedup vs baseline: 6.2618x; 1.6412x over previous
import functools

import jax
import jax.numpy as jnp
from jax import lax
from jax.experimental import pallas as pl
from jax.experimental.pallas import tpu as pltpu
from jax.experimental.pallas import tpu_sc as plsc

_K = 64
_NW = 16
_BLK = 400
_STRIP = 80
_WIN = 5120


def _tc_first(x, WT, a_l, a_r):
    n, d = x.shape
    h = WT.shape[1]

    def body(x_ref, w_ref, al_ref, ar_ref, wh_ref, s1_ref, s2_ref):
        wh = jnp.dot(x_ref[...], w_ref[...], preferred_element_type=jnp.float32)
        wh_ref[...] = wh
        s1_ref[...] = jnp.dot(wh, al_ref[...], preferred_element_type=jnp.float32)
        s2_ref[...] = jnp.dot(wh, ar_ref[...], preferred_element_type=jnp.float32)

    return pl.pallas_call(
        body,
        grid=(n // _BLK,),
        in_specs=[
            pl.BlockSpec((_BLK, d), lambda i: (i, 0)),
            pl.BlockSpec((d, h), lambda i: (0, 0)),
            pl.BlockSpec((h, 1), lambda i: (0, 0)),
            pl.BlockSpec((h, 1), lambda i: (0, 0)),
        ],
        out_specs=[
            pl.BlockSpec((_BLK, h), lambda i: (i, 0)),
            pl.BlockSpec((_BLK, 1), lambda i: (i, 0)),
            pl.BlockSpec((_BLK, 1), lambda i: (i, 0)),
        ],
        out_shape=[
            jax.ShapeDtypeStruct((n, h), jnp.float32),
            jax.ShapeDtypeStruct((n, 1), jnp.float32),
            jax.ShapeDtypeStruct((n, 1), jnp.float32),
        ],
    )(x, WT, a_l, a_r)


def _tc_post(acc, s_all):
    n, d = acc.shape
    nw = s_all.shape[0]
    s_t = s_all.T.reshape(n // _BLK, _BLK, nw)

    def body(acc_ref, s_ref, o_ref):
        num = acc_ref[...]
        den = jnp.sum(s_ref[0], axis=1)[:, None] + 1e-10
        o_ref[...] = jnp.maximum(num / den, 0.0)

    return pl.pallas_call(
        body,
        grid=(n // _BLK,),
        in_specs=[
            pl.BlockSpec((_BLK, d), lambda i: (i, 0)),
            pl.BlockSpec((1, _BLK, nw), lambda i: (i, 0, 0)),
        ],
        out_specs=pl.BlockSpec((_BLK, d), lambda i: (i, 0)),
        out_shape=jax.ShapeDtypeStruct((n, d), jnp.float32),
    )(acc, s_t)


def _sc_edge_pass(Wh, s1, s2, srcr, dstr, ntailg):
    n, d = Wh.shape
    nchunks = srcr.shape[1] * 2 // _K
    nfull = nchunks - 1
    nwin = (n + _WIN - 1) // _WIN
    mesh = plsc.VectorSubcoreMesh(
        core_axis_name="c", subcore_axis_name="s", num_cores=1)

    @functools.partial(
        pl.kernel,
        out_type=(
            jax.ShapeDtypeStruct((n, d), jnp.float32),
            jax.ShapeDtypeStruct((_NW, n), jnp.float32),
        ),
        mesh=mesh,
        compiler_params=pltpu.CompilerParams(needs_layout_passes=False),
        scratch_types=[
            pltpu.VMEM((n,), jnp.float32),
            pltpu.VMEM((n,), jnp.float32),
            pltpu.VMEM((n,), jnp.float32),
            pltpu.VMEM((nchunks * _K // 2,), jnp.int32),
            pltpu.VMEM((nchunks * _K // 2,), jnp.int32),
            pltpu.VMEM((1, _K), jnp.int32),
            pltpu.VMEM((1, _K), jnp.int32),
            pltpu.VMEM((1, _K), jnp.int32),
            pltpu.VMEM((1, _K), jnp.int32),
            pltpu.VMEM((1, 16 * ntailg), jnp.int32),
            pltpu.VMEM((_K, 128), jnp.float32),
            pltpu.VMEM((_K, 128), jnp.float32),
            pltpu.VMEM((16,), jnp.int32),
            pltpu.VMEM((16,), jnp.float32),
            pltpu.VMEM_SHARED((_WIN, 128), jnp.float32),
            pltpu.SemaphoreType.DMA,
            pltpu.SemaphoreType.DMA,
        ],
    )
    def k(wh_hbm, s1_hbm, s2_hbm, src_hbm, dst_hbm, outh_hbm, outs_hbm,
          s1_v, s2_v, sv_v, src_v, dst_v, dstw0, dstw1, scl0_v, scl1_v,
          scl2_v, rg0, rg1, tks, tcs, acc, sem0, sem1):
        sid = lax.axis_index("s")
        wid = sid
        pltpu.sync_copy(s1_hbm, s1_v)
        pltpu.sync_copy(s2_hbm, s2_v)
        pltpu.sync_copy(src_hbm.at[wid], src_v)
        pltpu.sync_copy(dst_hbm.at[wid], dst_v)

        zeros16 = jnp.zeros((16,), jnp.float32)

        def sv_zero(i, c):
            sv_v[pl.ds(pl.multiple_of(i * 16, 8), 16)] = zeros16
            return c

        lax.fori_loop(0, n // 16, sv_zero, 0)

        def rg_zero(i, c):
            for t in range(8):
                rg0[i, pl.ds(t * 16, 16)] = zeros16
            return c

        iota = lax.iota(jnp.int32, 16)
        ip = jnp.minimum(iota + 1, 15)
        im = jnp.maximum(iota - 1, 0)

        def build_dstw(j, dstw):
            for t in range(_K // 32):
                dw = dst_v[pl.ds(pl.multiple_of(j * (_K // 2) + t * 16, 16),
                                 16)]
                d32 = plsc.bitcast(dw, jnp.int16)
                da, db = plsc.unpack(d32, format=plsc.PackFormat.INTERLEAVED,
                                     preferred_element_type=jnp.int32)
                dstw[0, pl.ds(t * 32, 16)] = da
                dstw[0, pl.ds(t * 32 + 16, 16)] = db

        def make_group(w, lo, compute_s, idx_ref, rg, dstw):
            def group(j, g32):
                sw = src_v[pl.ds(pl.multiple_of(j * (_K // 2) + g32 // 2, 16),
                                 16)]
                s32 = plsc.bitcast(sw, jnp.int16)
                sva, svb = plsc.unpack(
                    s32, format=plsc.PackFormat.INTERLEAVED,
                    preferred_element_type=jnp.int32)
                for half, sv in ((0, sva), (1, svb)):
                    t16 = g32 + 16 * half
                    dv = dstw[0, pl.ds(t16, 16)]
                    z = (plsc.load_gather(s1_v, [sv])
                         + plsc.load_gather(s2_v, [dv]))
                    p = jnp.exp(jnp.minimum(-z, -0.2 * z))
                    if compute_s:
                        ks, vs = plsc.sort_key_val(sv, p)
                        cs = plsc.cumsum(vs)
                        tks[...] = ks
                        tcs[...] = cs
                        ks_next = plsc.load_gather(tks, [ip])
                        ks_prev = plsc.load_gather(tks, [im])
                        cs_prev = plsc.load_gather(tcs, [im])
                        is_end = (ks != ks_next) | (iota == 15)
                        is_start = (ks != ks_prev) & (iota > 0)
                        plsc.addupdate_scatter(sv_v, [ks], cs, mask=is_end)
                        plsc.addupdate_scatter(sv_v, [ks], -cs_prev,
                                               mask=is_start)
                    loc = sv - lo
                    valid = (loc >= 0) & (loc < _WIN)
                    p = jnp.where(valid, p, 0.0)
                    idx_ref[0, pl.ds(t16, 16)] = jnp.clip(loc, 0, _WIN - 1)
                    for r in range(16):
                        ps = p[r]
                        i = t16 + r
                        for u in range(8):
                            rg[i, pl.ds(u * 16, 16)] = (
                                rg[i, pl.ds(u * 16, 16)] * ps)
            return group

        npairs = nfull // 2

        for w in range(nwin):
            lo = w * _WIN
            wrows = min(n - lo, _WIN)
            lax.fori_loop(0, _K, rg_zero, 0)
            nz = _WIN // _K
            for q in range((nz + 15) // 16):
                idx = sid + 16 * q

                @pl.when(idx < nz)
                def _():
                    start = pl.multiple_of(idx * _K, 8)
                    pltpu.sync_copy(rg0, acc.at[pl.ds(start, _K)])

            plsc.subcore_barrier()

            group0 = make_group(w, lo, w == 0, scl0_v, rg0, dstw0)
            group1 = make_group(w, lo, w == 0, scl1_v, rg1, dstw1)
            groupt = make_group(w, lo, w == 0, scl2_v, rg0, dstw0)

            def compute(j, group):
                def tbody(t, c2):
                    group(j, pl.multiple_of(t * 32, 32))
                    return c2

                lax.fori_loop(0, _K // 32, tbody, 0)

            build_dstw(0, dstw0)
            pltpu.async_copy(wh_hbm.at[dstw0.at[0]], rg0, sem0)
            build_dstw(1, dstw1)
            pltpu.async_copy(wh_hbm.at[dstw1.at[0]], rg1, sem1)

            def pair(jj, c):
                j0 = jj * 2
                pltpu.make_async_copy(wh_hbm.at[dstw0.at[0]], rg0,
                                      sem0).wait()
                compute(j0, group0)
                pltpu.sync_copy(rg0, acc.at[scl0_v.at[0]], add=True)

                @pl.when(jj + 1 < npairs)
                def _():
                    build_dstw(j0 + 2, dstw0)
                    pltpu.async_copy(wh_hbm.at[dstw0.at[0]], rg0, sem0)

                pltpu.make_async_copy(wh_hbm.at[dstw1.at[0]], rg1,
                                      sem1).wait()
                compute(j0 + 1, group1)
                pltpu.sync_copy(rg1, acc.at[scl1_v.at[0]], add=True)

                @pl.when(jj + 1 < npairs)
                def _():
                    build_dstw(j0 + 3, dstw1)
                    pltpu.async_copy(wh_hbm.at[dstw1.at[0]], rg1, sem1)

                return c

            lax.fori_loop(0, npairs, pair, 0)

            build_dstw(nfull, dstw0)
            pltpu.async_copy(wh_hbm.at[dstw0.at[0]], rg0, sem0).wait()
            for t in range(ntailg // 2):
                groupt(nfull, t * 32)
            pltpu.sync_copy(rg0.at[pl.ds(0, 16 * ntailg)],
                            acc.at[scl2_v.at[0]], add=True)

            plsc.subcore_barrier()
            ndump = wrows // _STRIP
            for q in range((ndump + 15) // 16):
                idx = sid + 16 * q

                @pl.when(idx < ndump)
                def _():
                    start = pl.multiple_of(idx * _STRIP, 8)
                    pltpu.sync_copy(
                        acc.at[pl.ds(start, _STRIP)],
                        outh_hbm.at[pl.ds(pl.multiple_of(lo, 8) + start,
                                          _STRIP)])

            plsc.subcore_barrier()

        pltpu.sync_copy(sv_v, outs_hbm.at[wid])

    return k(Wh, s1, s2, srcr, dstr)


@jax.jit
def kernel(x, edge_index, W1, a1, W2, a2):
    n, d = x.shape
    e = edge_index.shape[1]
    h = W1.shape[0]
    o = W2.shape[0]
    eper = e // _NW
    nfull = eper // _K
    ntail = eper - nfull * _K
    ntailg = ntail // 16
    pad = _K - ntail
    ei16 = edge_index.astype(jnp.int16)
    src = jnp.pad(ei16[0].reshape(_NW, eper), ((0, 0), (0, pad)))
    dst = jnp.pad(ei16[1].reshape(_NW, eper), ((0, 0), (0, pad)))
    src = lax.bitcast_convert_type(src.reshape(_NW, -1, 2), jnp.int32)
    dst = lax.bitcast_convert_type(dst.reshape(_NW, -1, 2), jnp.int32)

    wt_stack = jnp.stack([W1.T, W2.T])
    al_stack = jnp.stack([a1[:, :h].T, a2[:, :o].T])
    ar_stack = jnp.stack([a1[:, h:].T, a2[:, o:].T])

    def layer_step(hcur, ws):
        wt, al, ar = ws
        Wh, s1, s2 = _tc_first(hcur, wt, al, ar)
        acc, sall = _sc_edge_pass(Wh, s1.reshape(n), s2.reshape(n), src, dst,
                                  ntailg)
        return _tc_post(acc, sall), None

    hfinal, _ = lax.scan(layer_step, x, (wt_stack, al_stack, ar_stack))
    return hfinal
